# baseline (speedup 1.0000x reference)
import functools

import jax
import jax.numpy as jnp
from jax import lax
from jax.experimental import pallas as pl
from jax.experimental.pallas import tpu as pltpu
from jax.experimental.pallas import tpu_sc as plsc

N_NODES = 5000
N_EDGES = 320000
IN_NF = 128
EDGE_NF = 16
H_NF = 256
OUT_NF = 128
EMB_NF = 4

_NC = 2
_NS = 16
_L = 16
_EC = N_EDGES // _NC
_B = 8000
_NBLK = _EC // _B
_N_PAD = 5120
_ACC = _L * _N_PAD


def _sc_segment_sum(idx_flat, val_flat):
    mesh = plsc.VectorSubcoreMesh(core_axis_name="c", subcore_axis_name="s")

    @functools.partial(
        pl.kernel,
        mesh=mesh,
        out_type=jax.ShapeDtypeStruct((_NC, EDGE_NF, _N_PAD), jnp.float32),
        compiler_params=pltpu.CompilerParams(use_tc_tiling_on_sc=False,
                                             needs_layout_passes=False),
        scratch_types=[
            pltpu.VMEM((_B,), jnp.int32),
            pltpu.VMEM((_B,), jnp.float32),
            pltpu.VMEM((_ACC,), jnp.float32),
            pltpu.VMEM((_N_PAD,), jnp.float32),
        ],
    )
    def k(idx_hbm, val_hbm, out_hbm, idx_v, val_v, acc_v, res_v):
        c = lax.axis_index("c")
        f = lax.axis_index("s")
        lane_off = lax.iota(jnp.int32, 16) * _N_PAD

        def zero_body(i, _):
            acc_v[pl.ds(i * 16, 16)] = jnp.zeros((16,), jnp.float32)
            return _
        lax.fori_loop(0, _ACC // 16, zero_body, 0)

        def blk_body(b, _):
            base = c * _EC + b * _B
            pltpu.sync_copy(idx_hbm.at[pl.ds(base, _B)], idx_v)
            pltpu.sync_copy(val_hbm.at[pl.ds(f * N_EDGES + base, _B)], val_v)

            def vec_body(k4, _):
                for u in range(4):
                    o = (k4 * 4 + u) * 16
                    iv = idx_v[pl.ds(o, 16)]
                    vv = val_v[pl.ds(o, 16)]
                    plsc.addupdate_scatter(acc_v, [iv + lane_off], vv)
                return _
            lax.fori_loop(0, _B // 64, vec_body, 0)
            return _
        lax.fori_loop(0, _NBLK, blk_body, 0)

        def red_body(n, _):
            o = n * 16
            s = acc_v[pl.ds(o, 16)]
            for r in range(1, 16):
                s = s + acc_v[pl.ds(r * _N_PAD + o, 16)]
            res_v[pl.ds(o, 16)] = s
            return _
        lax.fori_loop(0, _N_PAD // 16, red_body, 0)
        pltpu.sync_copy(res_v, out_hbm.at[c, f])

    return k(idx_flat, val_flat)


_DEC_T = 1024
_LOG2E = 1.4426950408889634


def _tc_body(nf_ref, p_ref, w1_ref, b1_ref, w2_ref, b2_ref, we_ref, be_ref,
             emb_ref, adj_ref, emb_s):
    i = pl.program_id(0)
    j = pl.program_id(1)

    @pl.when((i == 0) & (j == 0))
    def _mlp():
        agg_t = (p_ref[0] + p_ref[1])[:, :N_NODES]
        pre = (jnp.dot(nf_ref[...], w1_ref[0:IN_NF, :],
                       preferred_element_type=jnp.float32)
               + lax.dot_general(agg_t, w1_ref[IN_NF:IN_NF + EDGE_NF, :],
                                 (((0,), (0,)), ((), ())),
                                 preferred_element_type=jnp.float32)
               + b1_ref[...])
        h = jnp.maximum(pre, 0.0)
        out = (jnp.dot(h, w2_ref[...], preferred_element_type=jnp.float32)
               + b2_ref[...])
        emb = (jnp.dot(out, we_ref[...], preferred_element_type=jnp.float32)
               + be_ref[...])
        emb_s[pl.ds(0, N_NODES), :] = emb
        emb_ref[...] = emb

    xr = emb_s[pl.ds(i * _DEC_T, _DEC_T), :]
    xc = emb_s[pl.ds(j * _DEC_T, _DEC_T), :]
    rn = jnp.sum(xr * xr, axis=1, keepdims=True)
    cn = jnp.sum(xc * xc, axis=1, keepdims=True).reshape(1, _DEC_T)
    g = lax.dot_general(xr, xc, (((1,), (1,)), ((), ())),
                        preferred_element_type=jnp.float32)
    d2 = rn + cn - 2.0 * g
    e = jnp.exp2((1.0 - 3.0 * d2) * _LOG2E)
    a = 1.0 / (1.0 + e)

    @pl.when(i != j)
    def _off_diag():
        adj_ref[...] = a

    @pl.when(i == j)
    def _diag():
        rid = lax.broadcasted_iota(jnp.int32, (_DEC_T, _DEC_T), 0)
        cid = lax.broadcasted_iota(jnp.int32, (_DEC_T, _DEC_T), 1)
        adj_ref[...] = jnp.where(rid == cid, 0.0, a)


def kernel(node_feats, edge_index, edge_attr, W1, b1, W2, b2, W_emb, b_emb):
    idx_flat = edge_index[0]
    val_flat = edge_attr.T.reshape(EDGE_NF * N_EDGES)
    partials = _sc_segment_sum(idx_flat, val_flat)

    nt = pl.cdiv(N_NODES, _DEC_T)
    const = lambda i, j: (0, 0)
    const3 = lambda i, j: (0, 0, 0)
    node_emb, adj = pl.pallas_call(
        _tc_body,
        grid=(nt, nt),
        in_specs=[
            pl.BlockSpec((N_NODES, IN_NF), const),
            pl.BlockSpec((_NC, EDGE_NF, _N_PAD), const3),
            pl.BlockSpec((IN_NF + EDGE_NF, H_NF), const),
            pl.BlockSpec((1, H_NF), const),
            pl.BlockSpec((H_NF, OUT_NF), const),
            pl.BlockSpec((1, OUT_NF), const),
            pl.BlockSpec((OUT_NF, EMB_NF), const),
            pl.BlockSpec((1, EMB_NF), const),
        ],
        out_specs=[
            pl.BlockSpec((N_NODES, EMB_NF), const),
            pl.BlockSpec((_DEC_T, _DEC_T), lambda i, j: (i, j)),
        ],
        out_shape=[
            jax.ShapeDtypeStruct((N_NODES, EMB_NF), jnp.float32),
            jax.ShapeDtypeStruct((N_NODES, N_NODES), jnp.float32),
        ],
        scratch_shapes=[pltpu.VMEM((_N_PAD, EMB_NF), jnp.float32)],
    )(node_feats, partials, W1, b1.reshape(1, H_NF), W2,
      b2.reshape(1, OUT_NF), W_emb, b_emb.reshape(1, EMB_NF))

    return (node_emb, adj)


# speedup vs baseline: 1.7180x; 1.7180x over previous
import functools

import jax
import jax.numpy as jnp
from jax import lax
from jax.experimental import pallas as pl
from jax.experimental.pallas import tpu as pltpu
from jax.experimental.pallas import tpu_sc as plsc

N_NODES = 5000
N_EDGES = 320000
IN_NF = 128
EDGE_NF = 16
H_NF = 256
OUT_NF = 128
EMB_NF = 4

_NC = 2
_NS = 16
_L = 16
_EC = N_EDGES // _NC
_B = 6400
_NBLK = _EC // _B
_NBUF = 3
_UNROLL = 16
_N_PAD = 5120
_ACC = _L * _N_PAD


def _sc_segment_sum(idx_flat, val_flat):
    mesh = plsc.VectorSubcoreMesh(core_axis_name="c", subcore_axis_name="s")

    @functools.partial(
        pl.kernel,
        mesh=mesh,
        out_type=jax.ShapeDtypeStruct((_NC, EDGE_NF, _N_PAD), jnp.float32),
        compiler_params=pltpu.CompilerParams(use_tc_tiling_on_sc=False,
                                             needs_layout_passes=False),
        scratch_types=[
            pltpu.VMEM((_NBUF, _B), jnp.int32),
            pltpu.VMEM((_NBUF, _B), jnp.float32),
            pltpu.VMEM((_ACC,), jnp.float32),
            pltpu.VMEM((_N_PAD,), jnp.float32),
            pltpu.SemaphoreType.DMA,
            pltpu.SemaphoreType.DMA,
            pltpu.SemaphoreType.DMA,
        ],
    )
    def k(idx_hbm, val_hbm, out_hbm, idx_v, val_v, acc_v, res_v,
          sem0, sem1, sem2):
        c = lax.axis_index("c")
        f = lax.axis_index("s")
        sems = (sem0, sem1, sem2)
        lane_off = lax.iota(jnp.int32, 16) * _N_PAD

        def start(b, slot):
            base = c * _EC + b * _B
            hi = pltpu.async_copy(idx_hbm.at[pl.ds(base, _B)],
                                  idx_v.at[slot], sems[slot])
            hv = pltpu.async_copy(val_hbm.at[pl.ds(f * N_EDGES + base, _B)],
                                  val_v.at[slot], sems[slot])
            return hi, hv

        handles = [None] * _NBUF
        for b0 in range(_NBUF - 1):
            handles[b0] = start(b0, b0)

        def zero_body(i, _):
            for u in range(16):
                acc_v[pl.ds(i * 256 + u * 16, 16)] = jnp.zeros(
                    (16,), jnp.float32)
            return _
        lax.fori_loop(0, _ACC // 256, zero_body, 0)

        for b in range(_NBLK):
            slot = b % _NBUF
            if b + _NBUF - 1 < _NBLK:
                handles[(b + _NBUF - 1) % _NBUF] = start(
                    b + _NBUF - 1, (b + _NBUF - 1) % _NBUF)
            hi, hv = handles[slot]
            hi.wait()
            hv.wait()

            def vec_body(kk, _):
                o0 = kk * (_UNROLL * 16)
                ivs = [idx_v[slot, pl.ds(o0 + u * 16, 16)]
                       for u in range(_UNROLL)]
                vvs = [val_v[slot, pl.ds(o0 + u * 16, 16)]
                       for u in range(_UNROLL)]
                avs = [iv + lane_off for iv in ivs]
                for u in range(_UNROLL):
                    plsc.addupdate_scatter(acc_v, [avs[u]], vvs[u])
                return _
            lax.fori_loop(0, _B // (_UNROLL * 16), vec_body, 0)

        def red_body(n, _):
            o = n * 16
            vs = [acc_v[pl.ds(r * _N_PAD + o, 16)] for r in range(16)]
            while len(vs) > 1:
                vs = [vs[i] + vs[i + 1] for i in range(0, len(vs), 2)]
            res_v[pl.ds(o, 16)] = vs[0]
            return _
        lax.fori_loop(0, _N_PAD // 16, red_body, 0)
        pltpu.sync_copy(res_v, out_hbm.at[c, f])

    return k(idx_flat, val_flat)


_DEC_T = 1024
_LOG2E = 1.4426950408889634


def _tc_body(nf_ref, p_ref, w1_ref, b1_ref, w2_ref, b2_ref, we_ref, be_ref,
             emb_ref, adj_ref, emb_s):
    i = pl.program_id(0)
    j = pl.program_id(1)

    @pl.when((i == 0) & (j == 0))
    def _mlp():
        agg_t = (p_ref[0] + p_ref[1])[:, :N_NODES]
        pre = (jnp.dot(nf_ref[...], w1_ref[0:IN_NF, :],
                       preferred_element_type=jnp.float32)
               + lax.dot_general(agg_t, w1_ref[IN_NF:IN_NF + EDGE_NF, :],
                                 (((0,), (0,)), ((), ())),
                                 preferred_element_type=jnp.float32)
               + b1_ref[...])
        h = jnp.maximum(pre, 0.0)
        out = (jnp.dot(h, w2_ref[...], preferred_element_type=jnp.float32)
               + b2_ref[...])
        emb = (jnp.dot(out, we_ref[...], preferred_element_type=jnp.float32)
               + be_ref[...])
        emb_s[pl.ds(0, N_NODES), :] = emb
        emb_ref[...] = emb

    xr = emb_s[pl.ds(i * _DEC_T, _DEC_T), :]
    xc = emb_s[pl.ds(j * _DEC_T, _DEC_T), :]
    rn = jnp.sum(xr * xr, axis=1, keepdims=True)
    cn = jnp.sum(xc * xc, axis=1, keepdims=True).reshape(1, _DEC_T)
    g = lax.dot_general(xr, xc, (((1,), (1,)), ((), ())),
                        preferred_element_type=jnp.float32)
    d2 = rn + cn - 2.0 * g
    e = jnp.exp2((1.0 - 3.0 * d2) * _LOG2E)
    a = 1.0 / (1.0 + e)

    @pl.when(i != j)
    def _off_diag():
        adj_ref[...] = a

    @pl.when(i == j)
    def _diag():
        rid = lax.broadcasted_iota(jnp.int32, (_DEC_T, _DEC_T), 0)
        cid = lax.broadcasted_iota(jnp.int32, (_DEC_T, _DEC_T), 1)
        adj_ref[...] = jnp.where(rid == cid, 0.0, a)


def kernel(node_feats, edge_index, edge_attr, W1, b1, W2, b2, W_emb, b_emb):
    idx_flat = edge_index[0]
    val_flat = edge_attr.T.reshape(EDGE_NF * N_EDGES)
    partials = _sc_segment_sum(idx_flat, val_flat)

    nt = pl.cdiv(N_NODES, _DEC_T)
    const = lambda i, j: (0, 0)
    const3 = lambda i, j: (0, 0, 0)
    node_emb, adj = pl.pallas_call(
        _tc_body,
        grid=(nt, nt),
        in_specs=[
            pl.BlockSpec((N_NODES, IN_NF), const),
            pl.BlockSpec((_NC, EDGE_NF, _N_PAD), const3),
            pl.BlockSpec((IN_NF + EDGE_NF, H_NF), const),
            pl.BlockSpec((1, H_NF), const),
            pl.BlockSpec((H_NF, OUT_NF), const),
            pl.BlockSpec((1, OUT_NF), const),
            pl.BlockSpec((OUT_NF, EMB_NF), const),
            pl.BlockSpec((1, EMB_NF), const),
        ],
        out_specs=[
            pl.BlockSpec((N_NODES, EMB_NF), const),
            pl.BlockSpec((_DEC_T, _DEC_T), lambda i, j: (i, j)),
        ],
        out_shape=[
            jax.ShapeDtypeStruct((N_NODES, EMB_NF), jnp.float32),
            jax.ShapeDtypeStruct((N_NODES, N_NODES), jnp.float32),
        ],
        scratch_shapes=[pltpu.VMEM((_N_PAD, EMB_NF), jnp.float32)],
    )(node_feats, partials, W1, b1.reshape(1, H_NF), W2,
      b2.reshape(1, OUT_NF), W_emb, b_emb.reshape(1, EMB_NF))

    return (node_emb, adj)


# speedup vs baseline: 1.8970x; 1.1042x over previous
import functools

import jax
import jax.numpy as jnp
from jax import lax
from jax.experimental import pallas as pl
from jax.experimental.pallas import tpu as pltpu
from jax.experimental.pallas import tpu_sc as plsc

N_NODES = 5000
N_EDGES = 320000
IN_NF = 128
EDGE_NF = 16
H_NF = 256
OUT_NF = 128
EMB_NF = 4

_NC = 2
_NS = 16
_L = 16
_EC = N_EDGES // _NC
_B = 6400
_NBLK = _EC // _B
_NBUF = 3
_UNROLL = 16
_N_PAD = 5120
_ACC = _L * _N_PAD


def _sc_segment_sum(edge_index, val_flat):
    mesh = plsc.VectorSubcoreMesh(core_axis_name="c", subcore_axis_name="s")

    @functools.partial(
        pl.kernel,
        mesh=mesh,
        out_type=jax.ShapeDtypeStruct((_NC, EDGE_NF, _N_PAD), jnp.float32),
        compiler_params=pltpu.CompilerParams(use_tc_tiling_on_sc=False,
                                             needs_layout_passes=False),
        scratch_types=[
            pltpu.VMEM((_NBUF, _B), jnp.int32),
            pltpu.VMEM((_NBUF, _B), jnp.float32),
            pltpu.VMEM((_ACC,), jnp.float32),
            pltpu.VMEM((_N_PAD,), jnp.float32),
            pltpu.SemaphoreType.DMA,
            pltpu.SemaphoreType.DMA,
            pltpu.SemaphoreType.DMA,
        ],
    )
    def k(idx_hbm, val_hbm, out_hbm, idx_v, val_v, acc_v, res_v,
          sem0, sem1, sem2):
        c = lax.axis_index("c")
        f = lax.axis_index("s")
        sems = (sem0, sem1, sem2)
        lane_off = lax.iota(jnp.int32, 16) * _N_PAD

        def start(b, slot):
            base = c * _EC + b * _B
            hi = pltpu.async_copy(idx_hbm.at[0, pl.ds(base, _B)],
                                  idx_v.at[slot], sems[slot])
            hv = pltpu.async_copy(val_hbm.at[pl.ds(f * N_EDGES + base, _B)],
                                  val_v.at[slot], sems[slot])
            return hi, hv

        handles = [None] * _NBUF
        for b0 in range(_NBUF - 1):
            handles[b0] = start(b0, b0)

        def zero_body(i, _):
            for u in range(16):
                acc_v[pl.ds(i * 256 + u * 16, 16)] = jnp.zeros(
                    (16,), jnp.float32)
            return _
        lax.fori_loop(0, _ACC // 256, zero_body, 0)

        for b in range(_NBLK):
            slot = b % _NBUF
            if b + _NBUF - 1 < _NBLK:
                handles[(b + _NBUF - 1) % _NBUF] = start(
                    b + _NBUF - 1, (b + _NBUF - 1) % _NBUF)
            hi, hv = handles[slot]
            hi.wait()
            hv.wait()

            def vec_body(kk, _):
                o0 = kk * (_UNROLL * 16)
                ivs = [idx_v[slot, pl.ds(o0 + u * 16, 16)]
                       for u in range(_UNROLL)]
                vvs = [val_v[slot, pl.ds(o0 + u * 16, 16)]
                       for u in range(_UNROLL)]
                avs = [iv + lane_off for iv in ivs]
                for u in range(_UNROLL):
                    plsc.addupdate_scatter(acc_v, [avs[u]], vvs[u])
                return _
            lax.fori_loop(0, _B // (_UNROLL * 16), vec_body, 0)

        def red_body(n, _):
            o = n * 16
            vs = [acc_v[pl.ds(r * _N_PAD + o, 16)] for r in range(16)]
            while len(vs) > 1:
                vs = [vs[i] + vs[i + 1] for i in range(0, len(vs), 2)]
            res_v[pl.ds(o, 16)] = vs[0]
            return _
        lax.fori_loop(0, _N_PAD // 16, red_body, 0)
        pltpu.sync_copy(res_v, out_hbm.at[c, f])

    return k(edge_index, val_flat)


_DEC_T = 1024
_LOG2E = 1.4426950408889634


def _tc_body(nf_ref, p_ref, w1_ref, b1_ref, w2_ref, b2_ref, we_ref, be_ref,
             emb_ref, adj_ref, emb_s, emb6_s, ar_s, bc_s):
    i = pl.program_id(0)
    j = pl.program_id(1)

    @pl.when((i == 0) & (j == 0))
    def _mlp():
        agg_t = (p_ref[0] + p_ref[1])[:, :N_NODES]
        pre = (jnp.dot(nf_ref[...], w1_ref[0:IN_NF, :],
                       preferred_element_type=jnp.float32)
               + lax.dot_general(agg_t, w1_ref[IN_NF:IN_NF + EDGE_NF, :],
                                 (((0,), (0,)), ((), ())),
                                 preferred_element_type=jnp.float32)
               + b1_ref[...])
        h = jnp.maximum(pre, 0.0)
        out = (jnp.dot(h, w2_ref[...], preferred_element_type=jnp.float32)
               + b2_ref[...])
        emb = (jnp.dot(out, we_ref[...], preferred_element_type=jnp.float32)
               + be_ref[...])
        emb_s[pl.ds(0, N_NODES), :] = emb
        emb_ref[...] = emb
        emb6_s[pl.ds(0, N_NODES), :] = emb * (6.0 * _LOG2E)
        sq = jnp.sum(emb * emb, axis=1, keepdims=True)
        ar_s[pl.ds(0, N_NODES), :] = _LOG2E - (3.0 * _LOG2E) * sq
        bc_s[0:1, pl.ds(0, N_NODES)] = jnp.transpose(
            (-3.0 * _LOG2E) * sq, (1, 0))

    xr6 = emb6_s[pl.ds(i * _DEC_T, _DEC_T), :]
    xc = emb_s[pl.ds(j * _DEC_T, _DEC_T), :]
    gp = lax.dot_general(xr6, xc, (((1,), (1,)), ((), ())),
                         preferred_element_type=jnp.float32)
    t = (ar_s[pl.ds(i * _DEC_T, _DEC_T), :]
         + bc_s[0:1, pl.ds(j * _DEC_T, _DEC_T)] + gp)
    a = 1.0 / (1.0 + jnp.exp2(t))

    @pl.when(i != j)
    def _off_diag():
        adj_ref[...] = a

    @pl.when(i == j)
    def _diag():
        rid = lax.broadcasted_iota(jnp.int32, (_DEC_T, _DEC_T), 0)
        cid = lax.broadcasted_iota(jnp.int32, (_DEC_T, _DEC_T), 1)
        adj_ref[...] = jnp.where(rid == cid, 0.0, a)


def kernel(node_feats, edge_index, edge_attr, W1, b1, W2, b2, W_emb, b_emb):
    val_flat = edge_attr.T.reshape(EDGE_NF * N_EDGES)
    partials = _sc_segment_sum(edge_index, val_flat)

    nt = pl.cdiv(N_NODES, _DEC_T)
    const = lambda i, j: (0, 0)
    const3 = lambda i, j: (0, 0, 0)
    node_emb, adj = pl.pallas_call(
        _tc_body,
        grid=(nt, nt),
        in_specs=[
            pl.BlockSpec((N_NODES, IN_NF), const),
            pl.BlockSpec((_NC, EDGE_NF, _N_PAD), const3),
            pl.BlockSpec((IN_NF + EDGE_NF, H_NF), const),
            pl.BlockSpec((1, H_NF), const),
            pl.BlockSpec((H_NF, OUT_NF), const),
            pl.BlockSpec((1, OUT_NF), const),
            pl.BlockSpec((OUT_NF, EMB_NF), const),
            pl.BlockSpec((1, EMB_NF), const),
        ],
        out_specs=[
            pl.BlockSpec((N_NODES, EMB_NF), const),
            pl.BlockSpec((_DEC_T, _DEC_T), lambda i, j: (i, j)),
        ],
        out_shape=[
            jax.ShapeDtypeStruct((N_NODES, EMB_NF), jnp.float32),
            jax.ShapeDtypeStruct((N_NODES, N_NODES), jnp.float32),
        ],
        scratch_shapes=[pltpu.VMEM((_N_PAD, EMB_NF), jnp.float32),
                        pltpu.VMEM((_N_PAD, EMB_NF), jnp.float32),
                        pltpu.VMEM((_N_PAD, 1), jnp.float32),
                        pltpu.VMEM((8, _N_PAD), jnp.float32)],
    )(node_feats, partials, W1, b1.reshape(1, H_NF), W2,
      b2.reshape(1, OUT_NF), W_emb, b_emb.reshape(1, EMB_NF))

    return (node_emb, adj)


# speedup vs baseline: 2.1339x; 1.1249x over previous
import functools

import jax
import jax.numpy as jnp
from jax import lax
from jax.experimental import pallas as pl
from jax.experimental.pallas import tpu as pltpu
from jax.experimental.pallas import tpu_sc as plsc

N_NODES = 5000
N_EDGES = 320000
IN_NF = 128
EDGE_NF = 16
H_NF = 256
OUT_NF = 128
EMB_NF = 4

_NC = 2
_NS = 16
_L = 16
_EC = N_EDGES // _NC
_B = 6400
_NBLK = _EC // _B
_NBUF = 3
_UNROLL = 16
_N_PAD = 5120
_ACC = _L * _N_PAD


_NT = N_EDGES // 128


def _sc_segment_sum(idx3, val4):
    mesh = plsc.VectorSubcoreMesh(core_axis_name="c", subcore_axis_name="s")

    @functools.partial(
        pl.kernel,
        mesh=mesh,
        out_type=jax.ShapeDtypeStruct((_NC, EDGE_NF, _N_PAD), jnp.float32),
        compiler_params=pltpu.CompilerParams(use_tc_tiling_on_sc=False,
                                             needs_layout_passes=False),
        scratch_types=[
            pltpu.VMEM((_NBUF, _B // 128, 128), jnp.int32),
            pltpu.VMEM((_NBUF, _B // 128, 128), jnp.float32),
            pltpu.VMEM((_ACC,), jnp.float32),
            pltpu.VMEM((_N_PAD,), jnp.float32),
            pltpu.SemaphoreType.DMA,
            pltpu.SemaphoreType.DMA,
            pltpu.SemaphoreType.DMA,
        ],
    )
    def k(idx_hbm, val_hbm, out_hbm, idx_v, val_v, acc_v, res_v,
          sem0, sem1, sem2):
        c = lax.axis_index("c")
        f = lax.axis_index("s")
        g = f // 8
        r = f % 8
        sems = (sem0, sem1, sem2)
        lane_off = lax.iota(jnp.int32, 16) * _N_PAD

        def start(b, slot):
            tb = (c * _EC + b * _B) // 128
            hi = pltpu.async_copy(idx_hbm.at[pl.ds(tb, _B // 128), 0],
                                  idx_v.at[slot], sems[slot])
            hv = pltpu.async_copy(val_hbm.at[g, pl.ds(tb, _B // 128), r],
                                  val_v.at[slot], sems[slot])
            return hi, hv

        handles = [None] * _NBUF
        for b0 in range(_NBUF - 1):
            handles[b0] = start(b0, b0)

        def zero_body(i, _):
            for u in range(16):
                acc_v[pl.ds(i * 256 + u * 16, 16)] = jnp.zeros(
                    (16,), jnp.float32)
            return _
        lax.fori_loop(0, _ACC // 256, zero_body, 0)

        for b in range(_NBLK):
            slot = b % _NBUF
            if b + _NBUF - 1 < _NBLK:
                handles[(b + _NBUF - 1) % _NBUF] = start(
                    b + _NBUF - 1, (b + _NBUF - 1) % _NBUF)
            hi, hv = handles[slot]
            hi.wait()
            hv.wait()

            def vec_body(kk, _):
                r0 = kk * (_UNROLL // 8)
                ivs = [idx_v[slot, r0 + u // 8, pl.ds((u % 8) * 16, 16)]
                       for u in range(_UNROLL)]
                vvs = [val_v[slot, r0 + u // 8, pl.ds((u % 8) * 16, 16)]
                       for u in range(_UNROLL)]
                avs = [iv + lane_off for iv in ivs]
                for u in range(_UNROLL):
                    plsc.addupdate_scatter(acc_v, [avs[u]], vvs[u])
                return _
            lax.fori_loop(0, _B // (_UNROLL * 16), vec_body, 0)

        def red_body(n, _):
            o = n * 16
            vs = [acc_v[pl.ds(r * _N_PAD + o, 16)] for r in range(16)]
            while len(vs) > 1:
                vs = [vs[i] + vs[i + 1] for i in range(0, len(vs), 2)]
            res_v[pl.ds(o, 16)] = vs[0]
            return _
        lax.fori_loop(0, _N_PAD // 16, red_body, 0)
        pltpu.sync_copy(res_v, out_hbm.at[c, f])

    return k(idx3, val4)


_DEC_T = 1024
_LOG2E = 1.4426950408889634


def _tc_body(nf_ref, p_ref, w1_ref, b1_ref, w2_ref, b2_ref, we_ref, be_ref,
             emb_ref, adj_ref, emb_s, emb6_s, ar_s, bc_s):
    i = pl.program_id(0)
    j = pl.program_id(1)

    @pl.when((i == 0) & (j == 0))
    def _mlp():
        agg_t = (p_ref[0] + p_ref[1])[:, :N_NODES]
        pre = (jnp.dot(nf_ref[...], w1_ref[0:IN_NF, :],
                       preferred_element_type=jnp.float32)
               + lax.dot_general(agg_t, w1_ref[IN_NF:IN_NF + EDGE_NF, :],
                                 (((0,), (0,)), ((), ())),
                                 preferred_element_type=jnp.float32)
               + b1_ref[...])
        h = jnp.maximum(pre, 0.0)
        out = (jnp.dot(h, w2_ref[...], preferred_element_type=jnp.float32)
               + b2_ref[...])
        emb = (jnp.dot(out, we_ref[...], preferred_element_type=jnp.float32)
               + be_ref[...])
        emb_s[pl.ds(0, N_NODES), :] = emb
        emb_ref[...] = emb
        emb6_s[pl.ds(0, N_NODES), :] = emb * (6.0 * _LOG2E)
        sq = jnp.sum(emb * emb, axis=1, keepdims=True)
        ar_s[pl.ds(0, N_NODES), :] = _LOG2E - (3.0 * _LOG2E) * sq
        bc_s[0:1, pl.ds(0, N_NODES)] = jnp.transpose(
            (-3.0 * _LOG2E) * sq, (1, 0))

    xr6 = emb6_s[pl.ds(i * _DEC_T, _DEC_T), :]
    xc = emb_s[pl.ds(j * _DEC_T, _DEC_T), :]
    gp = lax.dot_general(xr6, xc, (((1,), (1,)), ((), ())),
                         preferred_element_type=jnp.float32)
    t = (ar_s[pl.ds(i * _DEC_T, _DEC_T), :]
         + bc_s[0:1, pl.ds(j * _DEC_T, _DEC_T)] + gp)
    a = 1.0 / (1.0 + jnp.exp2(t))

    @pl.when(i != j)
    def _off_diag():
        adj_ref[...] = a

    @pl.when(i == j)
    def _diag():
        rid = lax.broadcasted_iota(jnp.int32, (_DEC_T, _DEC_T), 0)
        cid = lax.broadcasted_iota(jnp.int32, (_DEC_T, _DEC_T), 1)
        adj_ref[...] = jnp.where(rid == cid, 0.0, a)


def kernel(node_feats, edge_index, edge_attr, W1, b1, W2, b2, W_emb, b_emb):
    idx3 = edge_index.reshape(2, _NT, 128).transpose(1, 0, 2)
    val4 = edge_attr.T.reshape(2, 8, _NT, 128).transpose(0, 2, 1, 3)
    partials = _sc_segment_sum(idx3, val4)

    nt = pl.cdiv(N_NODES, _DEC_T)
    const = lambda i, j: (0, 0)
    const3 = lambda i, j: (0, 0, 0)
    node_emb, adj = pl.pallas_call(
        _tc_body,
        grid=(nt, nt),
        in_specs=[
            pl.BlockSpec((N_NODES, IN_NF), const),
            pl.BlockSpec((_NC, EDGE_NF, _N_PAD), const3),
            pl.BlockSpec((IN_NF + EDGE_NF, H_NF), const),
            pl.BlockSpec((1, H_NF), const),
            pl.BlockSpec((H_NF, OUT_NF), const),
            pl.BlockSpec((1, OUT_NF), const),
            pl.BlockSpec((OUT_NF, EMB_NF), const),
            pl.BlockSpec((1, EMB_NF), const),
        ],
        out_specs=[
            pl.BlockSpec((N_NODES, EMB_NF), const),
            pl.BlockSpec((_DEC_T, _DEC_T), lambda i, j: (i, j)),
        ],
        out_shape=[
            jax.ShapeDtypeStruct((N_NODES, EMB_NF), jnp.float32),
            jax.ShapeDtypeStruct((N_NODES, N_NODES), jnp.float32),
        ],
        scratch_shapes=[pltpu.VMEM((_N_PAD, EMB_NF), jnp.float32),
                        pltpu.VMEM((_N_PAD, EMB_NF), jnp.float32),
                        pltpu.VMEM((_N_PAD, 1), jnp.float32),
                        pltpu.VMEM((8, _N_PAD), jnp.float32)],
    )(node_feats, partials, W1, b1.reshape(1, H_NF), W2,
      b2.reshape(1, OUT_NF), W_emb, b_emb.reshape(1, EMB_NF))

    return (node_emb, adj)
